# baseline (speedup 1.0000x reference)
import functools

import jax
import jax.numpy as jnp
from jax import lax
from jax.experimental import pallas as pl
from jax.experimental.pallas import tpu as pltpu
from jax.experimental.pallas import tpu_sc as plsc

DIM = 32
SUB = 128
CH = 5
NB = 4


@functools.cache
def _build(nrows: int, vocab: int):
    info = plsc.get_sparse_core_info()
    nw = info.num_cores * info.num_subcores
    rows_per_w = nrows // nw
    n_chunks = rows_per_w // CH
    assert rows_per_w % CH == 0 and n_chunks % NB == 0

    mesh = plsc.VectorSubcoreMesh(core_axis_name="c", subcore_axis_name="s")

    @functools.partial(
        pl.kernel,
        mesh=mesh,
        out_type=jax.ShapeDtypeStruct((nrows * SUB, DIM), jnp.float32),
        scratch_types=[
            pltpu.VMEM((rows_per_w * SUB,), jnp.int32),
            [pltpu.VMEM((CH * SUB, DIM), jnp.float32) for _ in range(NB)],
            [pltpu.SemaphoreType.DMA for _ in range(NB)],
            [pltpu.SemaphoreType.DMA for _ in range(NB)],
        ],
        compiler_params=pltpu.CompilerParams(use_tc_tiling_on_sc=False),
    )
    def gather_kernel(idx_hbm, table_hbm, out_hbm, idx_v, rows_v, sem_g, sem_o):
        wid = lax.axis_index("s") * info.num_cores + lax.axis_index("c")
        base = wid * rows_per_w
        pltpu.sync_copy(idx_hbm.at[pl.ds(base * SUB, rows_per_w * SUB)], idx_v)

        def outer_body(o, carry):
            for b in range(NB):
                c = o * NB + b

                @pl.when(o > 0)
                def _():
                    pltpu.make_async_copy(
                        rows_v[b], out_hbm.at[pl.ds(base, CH * SUB)], sem_o[b]
                    ).wait()

                pltpu.async_copy(
                    table_hbm.at[idx_v.at[pl.ds(c * (CH * SUB), CH * SUB)]],
                    rows_v[b],
                    sem_g[b],
                ).wait()
                pltpu.async_copy(
                    rows_v[b],
                    out_hbm.at[pl.ds((base + c * CH) * SUB, CH * SUB)],
                    sem_o[b],
                )
            return carry

        lax.fori_loop(0, n_chunks // NB, outer_body, 0)
        for b in range(NB):
            pltpu.make_async_copy(
                rows_v[b], out_hbm.at[pl.ds(base, CH * SUB)], sem_o[b]
            ).wait()

    return gather_kernel


def kernel(x, table):
    b, s = x.shape
    n = b * s
    nrows = n // SUB
    idx = x.reshape(-1).astype(jnp.int32)
    out = _build(nrows, table.shape[0])(idx, table)
    return out.reshape(b, s, DIM)


# speedup vs baseline: 1.0088x; 1.0088x over previous
import functools

import jax
import jax.numpy as jnp
from jax import lax
from jax.experimental import pallas as pl
from jax.experimental.pallas import tpu as pltpu
from jax.experimental.pallas import tpu_sc as plsc

DIM = 32
SUB = 128
CH = 10
NB = 2


@functools.cache
def _build(nrows: int, vocab: int):
    info = plsc.get_sparse_core_info()
    nw = info.num_cores * info.num_subcores
    rows_per_w = nrows // nw
    n_chunks = rows_per_w // CH
    assert rows_per_w % CH == 0 and n_chunks % NB == 0

    mesh = plsc.VectorSubcoreMesh(core_axis_name="c", subcore_axis_name="s")

    @functools.partial(
        pl.kernel,
        mesh=mesh,
        out_type=jax.ShapeDtypeStruct((nrows * SUB, DIM), jnp.float32),
        scratch_types=[
            pltpu.VMEM((rows_per_w * SUB,), jnp.int32),
            [pltpu.VMEM((CH * SUB, DIM), jnp.float32) for _ in range(NB)],
            [pltpu.SemaphoreType.DMA for _ in range(NB)],
            [pltpu.SemaphoreType.DMA for _ in range(NB)],
        ],
        compiler_params=pltpu.CompilerParams(use_tc_tiling_on_sc=False),
    )
    def gather_kernel(idx_hbm, table_hbm, out_hbm, idx_v, rows_v, sem_g, sem_o):
        wid = lax.axis_index("s") * info.num_cores + lax.axis_index("c")
        base = wid * rows_per_w
        pltpu.sync_copy(idx_hbm.at[pl.ds(base * SUB, rows_per_w * SUB)], idx_v)

        def outer_body(o, carry):
            for b in range(NB):
                c = o * NB + b

                @pl.when(o > 0)
                def _():
                    pltpu.make_async_copy(
                        rows_v[b], out_hbm.at[pl.ds(base, CH * SUB)], sem_o[b]
                    ).wait()

                pltpu.async_copy(
                    table_hbm.at[idx_v.at[pl.ds(c * (CH * SUB), CH * SUB)]],
                    rows_v[b],
                    sem_g[b],
                ).wait()
                pltpu.async_copy(
                    rows_v[b],
                    out_hbm.at[pl.ds((base + c * CH) * SUB, CH * SUB)],
                    sem_o[b],
                )
            return carry

        lax.fori_loop(0, (n_chunks - 2) // NB, outer_body, 0)
        c0 = n_chunks - 2
        pltpu.make_async_copy(
            rows_v[0], out_hbm.at[pl.ds(base, CH * SUB)], sem_o[0]
        ).wait()
        pltpu.async_copy(
            table_hbm.at[idx_v.at[pl.ds(c0 * (CH * SUB), CH * SUB)]],
            rows_v[0],
            sem_g[0],
        ).wait()
        pltpu.async_copy(
            rows_v[0],
            out_hbm.at[pl.ds((base + c0 * CH) * SUB, CH * SUB)],
            sem_o[0],
        )
        NP = 4
        P = CH * SUB // NP
        last = n_chunks - 1
        pltpu.make_async_copy(
            rows_v[1], out_hbm.at[pl.ds(base, CH * SUB)], sem_o[1]
        ).wait()
        for k in range(NP):
            off = last * (CH * SUB) + k * P
            pltpu.async_copy(
                table_hbm.at[idx_v.at[pl.ds(off, P)]],
                rows_v[1].at[pl.ds(k * P, P)],
                sem_g[1],
            ).wait()
            pltpu.async_copy(
                rows_v[1].at[pl.ds(k * P, P)],
                out_hbm.at[pl.ds(base * SUB + off, P)],
                sem_o[1],
            )
        pltpu.make_async_copy(
            rows_v[0], out_hbm.at[pl.ds(base, CH * SUB)], sem_o[0]
        ).wait()
        for k in range(NP):
            pltpu.make_async_copy(
                rows_v[1].at[pl.ds(k * P, P)],
                out_hbm.at[pl.ds(base, P)],
                sem_o[1],
            ).wait()

    return gather_kernel


def kernel(x, table):
    b, s = x.shape
    n = b * s
    nrows = n // SUB
    idx = x.reshape(-1).astype(jnp.int32)
    out = _build(nrows, table.shape[0])(idx, table)
    return out.reshape(b, s, DIM)


# speedup vs baseline: 1.0103x; 1.0014x over previous
import functools

import jax
import jax.numpy as jnp
from jax import lax
from jax.experimental import pallas as pl
from jax.experimental.pallas import tpu as pltpu
from jax.experimental.pallas import tpu_sc as plsc

DIM = 32
SUB = 128
CH = 10
NB = 2


@functools.cache
def _build(nrows: int, vocab: int):
    info = plsc.get_sparse_core_info()
    nw = info.num_cores * info.num_subcores
    rows_per_w = nrows // nw
    n_chunks = rows_per_w // CH
    assert rows_per_w % CH == 0 and n_chunks % NB == 0

    mesh = plsc.VectorSubcoreMesh(core_axis_name="c", subcore_axis_name="s")

    @functools.partial(
        pl.kernel,
        mesh=mesh,
        out_type=jax.ShapeDtypeStruct((nrows * SUB, DIM), jnp.float32),
        scratch_types=[
            pltpu.VMEM((rows_per_w * SUB,), jnp.int32),
            [pltpu.VMEM((CH * SUB, DIM), jnp.float32) for _ in range(NB)],
            [pltpu.SemaphoreType.DMA for _ in range(NB)],
            [pltpu.SemaphoreType.DMA for _ in range(NB)],
        ],
        compiler_params=pltpu.CompilerParams(use_tc_tiling_on_sc=False),
    )
    def gather_kernel(idx_hbm, table_hbm, out_hbm, idx_v, rows_v, sem_g, sem_o):
        wid = lax.axis_index("s") * info.num_cores + lax.axis_index("c")
        base = wid * rows_per_w
        pltpu.sync_copy(idx_hbm.at[pl.ds(base * SUB, rows_per_w * SUB)], idx_v)

        def outer_body(o, carry):
            for b in range(NB):
                c = o * NB + b

                @pl.when(o > 0)
                def _():
                    pltpu.make_async_copy(
                        rows_v[b], out_hbm.at[pl.ds(base, CH * SUB)], sem_o[b]
                    ).wait()

                pltpu.async_copy(
                    table_hbm.at[idx_v.at[pl.ds(c * (CH * SUB), CH * SUB)]],
                    rows_v[b],
                    sem_g[b],
                ).wait()
                pltpu.async_copy(
                    rows_v[b],
                    out_hbm.at[pl.ds((base + c * CH) * SUB, CH * SUB)],
                    sem_o[b],
                )
            return carry

        lax.fori_loop(0, n_chunks // NB, outer_body, 0)
        for b in range(NB):
            pltpu.make_async_copy(
                rows_v[b], out_hbm.at[pl.ds(base, CH * SUB)], sem_o[b]
            ).wait()

    return gather_kernel


def kernel(x, table):
    b, s = x.shape
    n = b * s
    nrows = n // SUB
    idx = x.reshape(-1).astype(jnp.int32)
    out = _build(nrows, table.shape[0])(idx, table)
    return out.reshape(b, s, DIM)
